# baseline (speedup 1.0000x reference)
import functools

import jax
import jax.numpy as jnp
from jax import lax
from jax.experimental import pallas as pl
from jax.experimental.pallas import tpu as pltpu
from jax.experimental.pallas import tpu_sc as plsc

_MODALITY_IDX = 2

_NC = 2
_NS = 16
_NW = _NC * _NS
_LANES = 16

_CHUNK_ROWS = 16


def kernel(key, val, device, modality_table):
    b, s, d = val.shape
    n = b * s
    total = n * d
    per_w = total // _NW
    chunk = _CHUNK_ROWS * d
    n_chunks = per_w // chunk
    assert per_w % chunk == 0

    flat = val.reshape(total)
    table_flat = modality_table.reshape(modality_table.size)
    mesh = plsc.VectorSubcoreMesh(core_axis_name="c", subcore_axis_name="s")

    @functools.partial(
        pl.kernel,
        out_type=jax.ShapeDtypeStruct((total,), jnp.float32),
        mesh=mesh,
        scratch_types=[
            pltpu.VMEM((chunk,), jnp.float32),
            pltpu.VMEM((chunk,), jnp.float32),
            pltpu.VMEM((chunk,), jnp.float32),
            pltpu.SemaphoreType.DMA,
            pltpu.SemaphoreType.DMA,
            pltpu.SemaphoreType.DMA,
            pltpu.SemaphoreType.DMA,
        ],
    )
    def sc_add(val_hbm, table_hbm, out_hbm, buf0, buf1, rep, in0, in1, o0, o1):
        wid = lax.axis_index("s") * _NC + lax.axis_index("c")
        base = wid * per_w

        for k in range(_CHUNK_ROWS):
            pltpu.sync_copy(
                table_hbm.at[pl.ds(_MODALITY_IDX * d, d)],
                rep.at[pl.ds(k * d, d)],
            )

        bufs = (buf0, buf1)
        in_sems = (in0, in1)
        out_sems = (o0, o1)

        def in_copy(i, bi):
            return pltpu.make_async_copy(
                val_hbm.at[pl.ds(base + i * chunk, chunk)], bufs[bi], in_sems[bi]
            )

        def out_copy(i, bi):
            return pltpu.make_async_copy(
                bufs[bi], out_hbm.at[pl.ds(base + i * chunk, chunk)], out_sems[bi]
            )

        def accum(bi):
            @plsc.parallel_loop(0, chunk, step=_LANES, unroll=8)
            def _(off):
                plsc.addupdate(
                    bufs[bi].at[pl.ds(off, _LANES)],
                    rep[pl.ds(off, _LANES)],
                )

        in_copy(0, 0).start()
        in_copy(1, 1).start()
        for i in range(n_chunks):
            bi = i % 2
            in_copy(i, bi).wait()
            accum(bi)
            out_copy(i, bi).start()
            if i + 2 < n_chunks:
                out_copy(i, bi).wait()
                in_copy(i + 2, bi).start()
        out_copy(n_chunks - 2, 0).wait()
        out_copy(n_chunks - 1, 1).wait()

    out = sc_add(flat, table_flat)
    return out.reshape(b, s, d)


# speedup vs baseline: 3.6473x; 3.6473x over previous
import functools

import jax
import jax.numpy as jnp
from jax import lax
from jax.experimental import pallas as pl
from jax.experimental.pallas import tpu as pltpu
from jax.experimental.pallas import tpu_sc as plsc

_MODALITY_IDX = 2

_NC = 2
_NS = 16

_BLOCK_ROWS = 2048


def _add_row_kernel(val_ref, row_ref, out_ref):
    out_ref[...] = val_ref[...] + row_ref[...]


def kernel(key, val, device, modality_table):
    b, s, d = val.shape
    n = b * s
    num_mod, _ = modality_table.shape
    table_flat = modality_table.reshape(num_mod * d)
    mesh = plsc.VectorSubcoreMesh(core_axis_name="c", subcore_axis_name="s")

    @functools.partial(
        pl.kernel,
        out_type=jax.ShapeDtypeStruct((d,), jnp.float32),
        mesh=mesh,
        scratch_types=[pltpu.VMEM((d,), jnp.float32)],
    )
    def sc_lookup(table_hbm, row_hbm, buf):
        wid = lax.axis_index("s") * _NC + lax.axis_index("c")

        @pl.when(wid == 0)
        def _():
            pltpu.sync_copy(table_hbm.at[pl.ds(_MODALITY_IDX * d, d)], buf)
            pltpu.sync_copy(buf, row_hbm)

    row = sc_lookup(table_flat).reshape(1, d)

    flat = val.reshape(n, d)
    out = pl.pallas_call(
        _add_row_kernel,
        grid=(n // _BLOCK_ROWS,),
        in_specs=[
            pl.BlockSpec((_BLOCK_ROWS, d), lambda i: (i, 0)),
            pl.BlockSpec((1, d), lambda i: (0, 0)),
        ],
        out_specs=pl.BlockSpec((_BLOCK_ROWS, d), lambda i: (i, 0)),
        out_shape=jax.ShapeDtypeStruct((n, d), val.dtype),
    )(flat, row)
    return out.reshape(b, s, d)


# speedup vs baseline: 3.7466x; 1.0272x over previous
import functools

import jax
import jax.numpy as jnp
from jax import lax
from jax.experimental import pallas as pl
from jax.experimental.pallas import tpu as pltpu
from jax.experimental.pallas import tpu_sc as plsc

_MODALITY_IDX = 2

_NC = 2
_NS = 16

_BLOCK_ROWS = 2048


def _add_row_kernel(val_ref, row_ref, out_ref):
    out_ref[...] = val_ref[...] + row_ref[...]


def kernel(key, val, device, modality_table):
    b, s, d = val.shape
    n = b * s
    num_mod, _ = modality_table.shape
    table_flat = modality_table.reshape(num_mod * d)
    mesh = plsc.ScalarSubcoreMesh(axis_name="c", num_cores=_NC)

    @functools.partial(
        pl.kernel,
        out_type=jax.ShapeDtypeStruct((d,), jnp.float32),
        mesh=mesh,
    )
    def sc_lookup(table_hbm, row_hbm):
        cid = lax.axis_index("c")

        @pl.when(cid == 0)
        def _():
            pltpu.sync_copy(table_hbm.at[pl.ds(_MODALITY_IDX * d, d)], row_hbm)

    row = sc_lookup(table_flat).reshape(1, d)

    flat = val.reshape(n, d)
    out = pl.pallas_call(
        _add_row_kernel,
        grid=(n // _BLOCK_ROWS,),
        in_specs=[
            pl.BlockSpec((_BLOCK_ROWS, d), lambda i: (i, 0)),
            pl.BlockSpec((1, d), lambda i: (0, 0)),
        ],
        out_specs=pl.BlockSpec((_BLOCK_ROWS, d), lambda i: (i, 0)),
        out_shape=jax.ShapeDtypeStruct((n, d), val.dtype),
    )(flat, row)
    return out.reshape(b, s, d)


# speedup vs baseline: 3.8561x; 1.0292x over previous
import functools

import jax
import jax.numpy as jnp
from jax import lax
from jax.experimental import pallas as pl
from jax.experimental.pallas import tpu as pltpu
from jax.experimental.pallas import tpu_sc as plsc

_MODALITY_IDX = 2

_NC = 2
_NS = 16

_BLOCK_ROWS = 2048


def _add_row_kernel(val_ref, row_ref, out_ref):
    out_ref[...] = val_ref[...] + row_ref[...]


def kernel(key, val, device, modality_table):
    b, s, d = val.shape
    n = b * s
    num_mod, _ = modality_table.shape
    table_flat = modality_table.reshape(num_mod * d)
    mesh = plsc.ScalarSubcoreMesh(axis_name="c", num_cores=1)

    @functools.partial(
        pl.kernel,
        out_type=jax.ShapeDtypeStruct((d,), jnp.float32),
        mesh=mesh,
    )
    def sc_lookup(table_hbm, row_hbm):
        pltpu.sync_copy(table_hbm.at[pl.ds(_MODALITY_IDX * d, d)], row_hbm)

    row = sc_lookup(table_flat).reshape(1, d)

    flat = val.reshape(n, d)
    out = pl.pallas_call(
        _add_row_kernel,
        grid=(n // _BLOCK_ROWS,),
        in_specs=[
            pl.BlockSpec((_BLOCK_ROWS, d), lambda i: (i, 0)),
            pl.BlockSpec((1, d), lambda i: (0, 0)),
        ],
        out_specs=pl.BlockSpec((_BLOCK_ROWS, d), lambda i: (i, 0)),
        out_shape=jax.ShapeDtypeStruct((n, d), val.dtype),
    )(flat, row)
    return out.reshape(b, s, d)
